# baseline (speedup 1.0000x reference)
import functools

import jax
import jax.numpy as jnp
from jax import lax
from jax.experimental import pallas as pl
from jax.experimental.pallas import tpu as pltpu
from jax.experimental.pallas import tpu_sc as plsc

_S = 2
_S3 = _S ** 3
_N = 50000
_D = 128
_C = 200
_NB = _N // _C
_NW = 32


def _sc_body(coords_hbm, feats_hbm, scalev_hbm, coords_out, feats_out,
             cbuf, fbuf, cobuf, svbuf, sem):
    wid = lax.axis_index("s") * 2 + lax.axis_index("c")
    nb_even = _NB // _NW
    n_extra = _NB - nb_even * _NW
    nb = nb_even + jnp.where(wid < n_extra, 1, 0)

    lane = lax.iota(jnp.int32, 16)
    f = lane & 3
    jA = lane >> 2
    jB = jA + 4
    pltpu.sync_copy(scalev_hbm, svbuf)
    s = svbuf[...]
    multv = jnp.where(f == 0, jnp.full((16,), 1, jnp.int32), s)
    zero = jnp.full((16,), 0, jnp.int32)

    def _off(j):
        return jnp.where(
            f == 0, zero,
            jnp.where(f == 1, (j >> 2) & 1,
                      jnp.where(f == 2, (j >> 1) & 1, j & 1)))

    offA = _off(jA)
    offB = _off(jB)
    dnums = lax.GatherDimensionNumbers(
        offset_dims=(), collapsed_slice_dims=(0,), start_index_map=(0,))

    def batch_body(t, carry):
        b = wid + _NW * t
        base = pl.multiple_of(b * _C, 8)
        pltpu.sync_copy(coords_hbm.at[pl.ds(base * 4, _C * 4)],
                        cbuf.at[pl.ds(0, _C * 4)])
        pltpu.sync_copy(feats_hbm.at[pl.ds(base, _C)], fbuf)

        outs = [
            pltpu.async_copy(
                fbuf, feats_out.at[pl.ds(base, _C), pl.ds(j * _D, _D)], sem)
            for j in range(_S3)
        ]

        def point_body(i, carry2):
            raw = cbuf[pl.ds(i * 4, 16)]
            g = lax.gather(raw, f[:, None], dimension_numbers=dnums,
                           slice_sizes=(1,),
                           mode=lax.GatherScatterMode.PROMISE_IN_BOUNDS)
            bv = g * multv
            cobuf[pl.ds(i * 32, 16)] = bv + offA
            cobuf[pl.ds(i * 32 + 16, 16)] = bv + offB
            return carry2

        lax.fori_loop(0, _C, point_body, 0)
        pltpu.sync_copy(cobuf, coords_out.at[pl.ds(base * 32, _C * 32)])
        for c in outs:
            c.wait()
        return carry

    lax.fori_loop(0, nb, batch_body, 0)


_sc_call = functools.partial(
    pl.kernel,
    out_type=[
        jax.ShapeDtypeStruct((_N * 32,), jnp.int32),
        jax.ShapeDtypeStruct((_N, _S3 * _D), jnp.float32),
    ],
    mesh=plsc.VectorSubcoreMesh(core_axis_name="c", subcore_axis_name="s"),
    scratch_types=[
        pltpu.VMEM((_C * 4 + 16,), jnp.int32),
        pltpu.VMEM((_C, _D), jnp.float32),
        pltpu.VMEM((_C * 32,), jnp.int32),
        pltpu.VMEM((16,), jnp.int32),
        pltpu.SemaphoreType.DMA,
    ],
)(_sc_body)


def kernel(coords, feats, scale):
    N, d = feats.shape
    scale_v = jnp.full((16,), scale, jnp.int32)
    co, fo = _sc_call(coords.reshape(-1), feats, scale_v)
    return co.reshape(N * _S3, 4), fo.reshape(N * _S3, d)


# speedup vs baseline: 1.2443x; 1.2443x over previous
import jax
import jax.numpy as jnp
from jax.experimental import pallas as pl
from jax.experimental.pallas import tpu as pltpu

_S = 2
_S3 = _S ** 3
_D = 128


def _body(scale_ref, coords_ref, feats_ref, coords_out_ref, feats_out_ref,
          sem):
    i = pl.program_id(0)
    B = feats_ref.shape[0]
    copies = [
        pltpu.make_async_copy(
            feats_ref,
            feats_out_ref.at[pl.ds(i * B, B), pl.ds(j * _D, _D)],
            sem)
        for j in range(_S3)
    ]
    for c in copies:
        c.start()

    c = coords_ref[...]
    s = scale_ref[0]
    b = c[:, 0:1]
    x = c[:, 1:2] * s
    y = c[:, 2:3] * s
    z = c[:, 3:4] * s
    p = jax.lax.broadcasted_iota(jnp.int32, (B, 4 * _S3), 1)
    fld = p & 3
    j = p >> 2
    out = jnp.where(
        fld == 0, b,
        jnp.where(fld == 1, x + ((j >> 2) & 1),
                  jnp.where(fld == 2, y + ((j >> 1) & 1), z + (j & 1))))
    coords_out_ref[...] = out

    for c_ in copies:
        c_.wait()


def kernel(coords, feats, scale):
    N, d = feats.shape
    B = 2000
    grid = (N // B,)
    scale_arr = jnp.asarray(scale, jnp.int32).reshape(1)
    coords_out, feats_out = pl.pallas_call(
        _body,
        grid=grid,
        in_specs=[
            pl.BlockSpec(memory_space=pltpu.SMEM),
            pl.BlockSpec((B, 4), lambda i: (i, 0)),
            pl.BlockSpec((B, d), lambda i: (i, 0)),
        ],
        out_specs=[
            pl.BlockSpec((B, 4 * _S3), lambda i: (i, 0)),
            pl.BlockSpec(memory_space=pl.ANY),
        ],
        out_shape=[
            jax.ShapeDtypeStruct((N, 4 * _S3), jnp.int32),
            jax.ShapeDtypeStruct((N, _S3 * d), jnp.float32),
        ],
        scratch_shapes=[pltpu.SemaphoreType.DMA],
    )(scale_arr, coords, feats)
    return coords_out.reshape(N * _S3, 4), feats_out.reshape(N * _S3, d)


# speedup vs baseline: 2.1690x; 1.7432x over previous
import jax
import jax.numpy as jnp
from jax import lax
from jax.experimental import pallas as pl
from jax.experimental.pallas import tpu as pltpu

_S = 2
_S3 = _S ** 3
_D = 128


def _body(scale_ref, coords_ref, feats_ref, coords_out_ref, feats_out_ref):
    f = feats_ref[...]
    B, d = f.shape
    feats_out_ref[...] = jnp.broadcast_to(f[:, None, :], (B, _S3, d))

    c = coords_ref[...]
    s = scale_ref[0]
    c_rep = lax.broadcast_in_dim(c, (B, _S3, 4), (0, 2)).reshape(B * _S3, 4)
    r = lax.broadcasted_iota(jnp.int32, (B * _S3, 4), 0)
    k = lax.broadcasted_iota(jnp.int32, (B * _S3, 4), 1)
    j = r & 7
    mult = jnp.where(k == 0, 1, s)
    off = jnp.where(
        k == 0, 0,
        jnp.where(k == 1, (j >> 2) & 1,
                  jnp.where(k == 2, (j >> 1) & 1, j & 1)))
    coords_out_ref[...] = c_rep * mult + off


def kernel(coords, feats, scale):
    N, d = feats.shape
    B = 1000
    grid = (N // B,)
    scale_arr = jnp.asarray(scale, jnp.int32).reshape(1)
    coords_out, feats_out = pl.pallas_call(
        _body,
        grid=grid,
        in_specs=[
            pl.BlockSpec(memory_space=pltpu.SMEM),
            pl.BlockSpec((B, 4), lambda i: (i, 0)),
            pl.BlockSpec((B, d), lambda i: (i, 0)),
        ],
        out_specs=[
            pl.BlockSpec((B * _S3, 4), lambda i: (i, 0)),
            pl.BlockSpec((B, _S3, d), lambda i: (i, 0, 0)),
        ],
        out_shape=[
            jax.ShapeDtypeStruct((N * _S3, 4), jnp.int32),
            jax.ShapeDtypeStruct((N, _S3, d), jnp.float32),
        ],
    )(scale_arr, coords, feats)
    return coords_out, feats_out.reshape(N * _S3, d)


# speedup vs baseline: 2.1960x; 1.0124x over previous
import jax
import jax.numpy as jnp
from jax import lax
from jax.experimental import pallas as pl
from jax.experimental.pallas import tpu as pltpu

_S = 2
_S3 = _S ** 3
_D = 128


def _body(scale_ref, coords_ref, feats_ref, coords_out_ref, feats_out_ref):
    f = feats_ref[...]
    B, d = f.shape
    feats_out_ref[...] = jnp.broadcast_to(f[:, None, :], (B, _S3, d))

    c = coords_ref[...]
    s = scale_ref[0]
    c_rep = lax.broadcast_in_dim(c, (B, _S3, 4), (0, 2)).reshape(B * _S3, 4)
    r = lax.broadcasted_iota(jnp.int32, (B * _S3, 4), 0)
    k = lax.broadcasted_iota(jnp.int32, (B * _S3, 4), 1)
    j = r & 7
    mult = jnp.where(k == 0, 1, s)
    off = jnp.where(
        k == 0, 0,
        jnp.where(k == 1, (j >> 2) & 1,
                  jnp.where(k == 2, (j >> 1) & 1, j & 1)))
    coords_out_ref[...] = c_rep * mult + off


def kernel(coords, feats, scale):
    N, d = feats.shape
    B = 2000
    grid = (N // B,)
    scale_arr = jnp.asarray(scale, jnp.int32).reshape(1)
    coords_out, feats_out = pl.pallas_call(
        _body,
        grid=grid,
        in_specs=[
            pl.BlockSpec(memory_space=pltpu.SMEM),
            pl.BlockSpec((B, 4), lambda i: (i, 0)),
            pl.BlockSpec((B, d), lambda i: (i, 0)),
        ],
        out_specs=[
            pl.BlockSpec((B * _S3, 4), lambda i: (i, 0)),
            pl.BlockSpec((B, _S3, d), lambda i: (i, 0, 0)),
        ],
        out_shape=[
            jax.ShapeDtypeStruct((N * _S3, 4), jnp.int32),
            jax.ShapeDtypeStruct((N, _S3, d), jnp.float32),
        ],
    )(scale_arr, coords, feats)
    return coords_out, feats_out.reshape(N * _S3, d)
